# baseline (speedup 1.0000x reference)
import functools

import jax
import jax.numpy as jnp
from jax import lax
from jax.experimental import pallas as pl
from jax.experimental.pallas import tpu as pltpu
from jax.experimental.pallas import tpu_sc as plsc

B = 4
E = 32
N = 64 * 64 * 64
V = 1000


def _make_kernel():
    info = plsc.get_sparse_core_info()
    nc, ns, nl = info.num_cores, info.num_subcores, info.num_lanes
    nw = nc * ns
    wpb = nw // B
    ch = N // wpb

    mesh = plsc.VectorSubcoreMesh(core_axis_name="c", subcore_axis_name="s")

    @functools.partial(
        pl.kernel,
        out_type=jax.ShapeDtypeStruct((B, E, N), jnp.float32),
        mesh=mesh,
        compiler_params=pltpu.CompilerParams(needs_layout_passes=False),
        scratch_types=[
            pltpu.VMEM((ch,), jnp.int32),
            pltpu.VMEM((E * V,), jnp.float32),
            pltpu.VMEM((ch,), jnp.float32),
            pltpu.VMEM((ch,), jnp.float32),
            pltpu.SemaphoreType.DMA,
            pltpu.SemaphoreType.DMA,
        ],
    )
    def emb(idx_hbm, tab_hbm, out_hbm, idx_v, tab_v, buf_a, buf_b, sem_a, sem_b):
        wid = lax.axis_index("s") * nc + lax.axis_index("c")
        b = wid // wpb
        base = (wid % wpb) * ch

        pltpu.sync_copy(tab_hbm, tab_v)
        pltpu.sync_copy(idx_hbm.at[b, pl.ds(base, ch)], idx_v)

        bufs = (buf_a, buf_b)
        sems = (sem_a, sem_b)
        copies = [None, None]
        for e in range(E):
            k = e % 2
            if copies[k] is not None:
                copies[k].wait()
            buf = bufs[k]

            col = tab_v.at[pl.ds(e * V, V)]

            @plsc.parallel_loop(0, ch, step=nl, unroll=8)
            def v_body(i):
                sl = pl.ds(i, nl)
                idx = idx_v[sl]
                buf[sl] = plsc.load_gather(col, [idx])

            copies[k] = pltpu.async_copy(
                buf, out_hbm.at[b, e, pl.ds(base, ch)], sems[k]
            )
        copies[0].wait()
        copies[1].wait()

    return emb


def kernel(x, table):
    xi = x.reshape(B, N).astype(jnp.int32)
    tab_t = table.T.reshape(-1)
    out = _make_kernel()(xi, tab_t)
    return out.reshape(B, E, 64, 64, 64)


# speedup vs baseline: 2.7565x; 2.7565x over previous
import functools

import jax
import jax.numpy as jnp
from jax import lax
from jax.experimental import pallas as pl
from jax.experimental.pallas import tpu as pltpu
from jax.experimental.pallas import tpu_sc as plsc

B = 4
E = 32
D = 64
N = D * D * D
V = 1000
SLAB = 4


def _make_kernel():
    info = plsc.get_sparse_core_info()
    nc, ns, nl = info.num_cores, info.num_subcores, info.num_lanes
    nw = nc * ns
    wpb = nw // B
    ch = N // wpb
    dpw = D // wpb
    sub = SLAB * D * D

    mesh = plsc.VectorSubcoreMesh(core_axis_name="c", subcore_axis_name="s")

    @functools.partial(
        pl.kernel,
        out_type=jax.ShapeDtypeStruct((B * E * D, D, D), jnp.float32),
        mesh=mesh,
        compiler_params=pltpu.CompilerParams(needs_layout_passes=False),
        scratch_types=[
            pltpu.VMEM((ch,), jnp.int32),
            pltpu.VMEM((E * V,), jnp.float32),
            pltpu.VMEM((SLAB, D, D), jnp.float32),
            pltpu.VMEM((SLAB, D, D), jnp.float32),
            pltpu.SemaphoreType.DMA,
            pltpu.SemaphoreType.DMA,
        ],
    )
    def emb(idx_hbm, tab_hbm, out_hbm, idx_v, tab_v, buf_a, buf_b, sem_a, sem_b):
        wid = lax.axis_index("s") * nc + lax.axis_index("c")
        b = wid // wpb
        base = (wid % wpb) * ch
        d0 = (wid % wpb) * dpw

        pltpu.sync_copy(tab_hbm, tab_v)
        pltpu.sync_copy(idx_hbm.at[b, pl.ds(base, ch)], idx_v)

        bufs = (buf_a, buf_b)
        sems = (sem_a, sem_b)
        copies = [None, None]
        for s in range(E * dpw // SLAB):
            e = s // (dpw // SLAB)
            half = s % (dpw // SLAB)
            k = s % 2
            if copies[k] is not None:
                copies[k].wait()
            buf = bufs[k]
            col = tab_v.at[pl.ds(e * V, V)]
            ibase = half * sub

            @plsc.parallel_loop(0, sub, step=D, unroll=2)
            def row_body(i):
                d = lax.shift_right_logical(i, 12)
                h = lax.bitwise_and(lax.shift_right_logical(i, 6), D - 1)
                for w4 in range(D // nl):
                    idx = idx_v[pl.ds(ibase + i + w4 * nl, nl)]
                    buf[d, h, pl.ds(w4 * nl, nl)] = plsc.load_gather(col, [idx])

            g = (b * E + e) * D + d0 + half * SLAB
            copies[k] = pltpu.async_copy(
                buf, out_hbm.at[pl.ds(g, SLAB)], sems[k]
            )
        copies[0].wait()
        copies[1].wait()

    return emb


def kernel(x, table):
    xi = x.reshape(B, N).astype(jnp.int32)
    tab_t = table.T.reshape(-1)
    out = _make_kernel()(xi, tab_t)
    return out.reshape(B, E, D, D, D)
